# baseline (speedup 1.0000x reference)
import functools

import jax
import jax.numpy as jnp
from jax import lax
from jax.experimental import pallas as pl
from jax.experimental.pallas import tpu as pltpu
from jax.experimental.pallas import tpu_sc as plsc

N_NODES = 10000
N_EDGES = 320000
D_FEAT = 128
NUM_CORES = 2
NUM_SUBCORES = 16
NUM_WORKERS = NUM_CORES * NUM_SUBCORES
EDGES_PER_WORKER = N_EDGES // NUM_WORKERS
CHUNK = 80
NUM_CHUNKS = EDGES_PER_WORKER // CHUNK
GROUPS = CHUNK // 16
WORDS = D_FEAT // 2
ROWS_PER_PACKER = N_NODES // NUM_SUBCORES


def _pack_body(s_hbm, t_hbm, sp_hbm, tp_hbm, stage, packed, sem):
    wid = lax.axis_index("s") * NUM_CORES + lax.axis_index("c")

    def pack_one(src, dst, row0):
        pltpu.async_copy(src.at[pl.ds(row0, ROWS_PER_PACKER)], stage, sem).wait()

        def row_body(r, carry):
            for k in range(D_FEAT // 32):
                a = stage[r, pl.ds(k * 32, 16)]
                c = stage[r, pl.ds(k * 32 + 16, 16)]
                pk = plsc.pack(a, c, format=plsc.PackFormat.INTERLEAVED)
                packed[r, pl.ds(k * 16, 16)] = plsc.bitcast(pk, jnp.int32)
            return carry

        lax.fori_loop(0, ROWS_PER_PACKER, row_body, 0)
        pltpu.async_copy(packed, dst.at[pl.ds(row0, ROWS_PER_PACKER)], sem).wait()

    half = wid // NUM_SUBCORES
    row0 = (wid % NUM_SUBCORES) * ROWS_PER_PACKER

    @pl.when(half == 0)
    def _():
        pack_one(s_hbm, sp_hbm, row0)

    @pl.when(half == 1)
    def _():
        pack_one(t_hbm, tp_hbm, row0)


def _decoder_body(s_hbm, t_hbm, ei_hbm, out_hbm,
                  sidx, didx, srows, trows, outv,
                  sem_s0, sem_s1, sem_t0, sem_t1):
    wid = lax.axis_index("s") * NUM_CORES + lax.axis_index("c")
    base = wid * EDGES_PER_WORKER
    pltpu.sync_copy(ei_hbm.at[0, pl.ds(base, EDGES_PER_WORKER)], sidx)
    pltpu.sync_copy(ei_hbm.at[1, pl.ds(base, EDGES_PER_WORKER)], didx)
    lanes = lax.iota(jnp.int32, 16)
    sem_s = (sem_s0, sem_s1)
    sem_t = (sem_t0, sem_t1)

    def gather_start(ci, b):
        off = pl.multiple_of(ci * CHUNK, 8)
        pltpu.async_copy(s_hbm.at[sidx.at[pl.ds(off, CHUNK)]], srows.at[b], sem_s[b])
        pltpu.async_copy(t_hbm.at[didx.at[pl.ds(off, CHUNK)]], trows.at[b], sem_t[b])

    def gather_wait(b):
        pltpu.make_async_copy(s_hbm.at[pl.ds(0, CHUNK)], srows.at[b], sem_s[b]).wait()
        pltpu.make_async_copy(t_hbm.at[pl.ds(0, CHUNK)], trows.at[b], sem_t[b]).wait()

    def compute(ci, b):
        off = ci * CHUNK

        for gi in range(GROUPS):
            e0 = gi * 16
            vec = jnp.zeros((16,), jnp.float32)
            for j in range(16):
                e = e0 + j
                acc = jnp.zeros((16,), jnp.float32)
                for k in range(WORDS // 16):
                    sw = plsc.bitcast(srows[b, e, pl.ds(k * 16, 16)], jnp.bfloat16)
                    tw = plsc.bitcast(trows[b, e, pl.ds(k * 16, 16)], jnp.bfloat16)
                    plo, phi = plsc.unpack(sw * tw, format=plsc.PackFormat.INTERLEAVED)
                    acc = acc + plo + phi
                vec = jnp.where(lanes == j, jnp.sum(acc), vec)
            outv[pl.ds(off + e0, 16)] = vec

    gather_start(0, 0)
    gather_start(1, 1)

    def pair_body(p, carry):
        ci0 = 2 * p
        for b in range(2):
            ci = ci0 + b
            gather_wait(b)
            compute(ci, b)

            @pl.when(ci + 2 < NUM_CHUNKS)
            def _():
                gather_start(ci + 2, b)
        return carry

    lax.fori_loop(0, NUM_CHUNKS // 2, pair_body, 0)
    gather_wait(0)
    compute(NUM_CHUNKS - 1, 0)
    pltpu.sync_copy(outv, out_hbm.at[pl.ds(base, EDGES_PER_WORKER)])


@functools.partial(jax.jit)
def kernel(s, t, edge_index):
    ei = edge_index.astype(jnp.int32)
    mesh = plsc.VectorSubcoreMesh(core_axis_name="c", subcore_axis_name="s")
    params = pltpu.CompilerParams(needs_layout_passes=False,
                                  use_tc_tiling_on_sc=False)
    pack = pl.kernel(
        _pack_body,
        out_type=(jax.ShapeDtypeStruct((N_NODES, WORDS), jnp.int32),
                  jax.ShapeDtypeStruct((N_NODES, WORDS), jnp.int32)),
        mesh=mesh,
        compiler_params=params,
        scratch_types=[
            pltpu.VMEM((ROWS_PER_PACKER, D_FEAT), jnp.float32),
            pltpu.VMEM((ROWS_PER_PACKER, WORDS), jnp.int32),
            pltpu.SemaphoreType.DMA,
        ],
    )
    sp, tp = pack(s, t)
    run = pl.kernel(
        _decoder_body,
        out_type=jax.ShapeDtypeStruct((N_EDGES,), jnp.float32),
        mesh=mesh,
        compiler_params=params,
        scratch_types=[
            pltpu.VMEM((EDGES_PER_WORKER,), jnp.int32),
            pltpu.VMEM((EDGES_PER_WORKER,), jnp.int32),
            pltpu.VMEM((2, CHUNK, WORDS), jnp.int32),
            pltpu.VMEM((2, CHUNK, WORDS), jnp.int32),
            pltpu.VMEM((EDGES_PER_WORKER,), jnp.float32),
            pltpu.SemaphoreType.DMA,
            pltpu.SemaphoreType.DMA,
            pltpu.SemaphoreType.DMA,
            pltpu.SemaphoreType.DMA,
        ],
    )
    return run(sp, tp, ei)


# speedup vs baseline: 1.4356x; 1.4356x over previous
import functools

import jax
import jax.numpy as jnp
from jax import lax
from jax.experimental import pallas as pl
from jax.experimental.pallas import tpu as pltpu
from jax.experimental.pallas import tpu_sc as plsc

N_NODES = 10000
N_EDGES = 320000
D_FEAT = 128
NUM_CORES = 2
NUM_SUBCORES = 16
NUM_WORKERS = NUM_CORES * NUM_SUBCORES
EDGES_PER_WORKER = N_EDGES // NUM_WORKERS
CHUNK = 80
NUM_CHUNKS = EDGES_PER_WORKER // CHUNK
GROUPS = CHUNK // 16
WORDS = D_FEAT // 2
ROWS_PER_PACKER = N_NODES // NUM_SUBCORES


def _pack_body(s_hbm, t_hbm, sp_hbm, tp_hbm, stage, packed, sem):
    wid = lax.axis_index("s") * NUM_CORES + lax.axis_index("c")

    def pack_one(src, dst, row0):
        pltpu.async_copy(src.at[pl.ds(row0, ROWS_PER_PACKER)], stage, sem).wait()

        def row_body(r, carry):
            for k in range(D_FEAT // 32):
                a = stage[r, pl.ds(k * 32, 16)]
                c = stage[r, pl.ds(k * 32 + 16, 16)]
                pk = plsc.pack(a, c, format=plsc.PackFormat.INTERLEAVED)
                packed[r, pl.ds(k * 16, 16)] = plsc.bitcast(pk, jnp.int32)
            return carry

        lax.fori_loop(0, ROWS_PER_PACKER, row_body, 0)
        pltpu.async_copy(packed, dst.at[pl.ds(row0, ROWS_PER_PACKER)], sem).wait()

    half = wid // NUM_SUBCORES
    row0 = (wid % NUM_SUBCORES) * ROWS_PER_PACKER

    @pl.when(half == 0)
    def _():
        pack_one(s_hbm, sp_hbm, row0)

    @pl.when(half == 1)
    def _():
        pack_one(t_hbm, tp_hbm, row0)


def _decoder_body(s_hbm, t_hbm, ei_hbm, out_hbm,
                  sidx, didx, srows, trows, outv, sem_s, sem_t):
    wid = lax.axis_index("s") * NUM_CORES + lax.axis_index("c")
    base = wid * EDGES_PER_WORKER
    pltpu.sync_copy(ei_hbm.at[0, pl.ds(base, EDGES_PER_WORKER)], sidx)
    pltpu.sync_copy(ei_hbm.at[1, pl.ds(base, EDGES_PER_WORKER)], didx)
    lanes = lax.iota(jnp.int32, 16)

    def gather_start(ci, b):
        off = pl.multiple_of(ci * CHUNK, 8)
        pltpu.async_copy(s_hbm.at[sidx.at[pl.ds(off, CHUNK)]], srows.at[b], sem_s.at[b])
        pltpu.async_copy(t_hbm.at[didx.at[pl.ds(off, CHUNK)]], trows.at[b], sem_t.at[b])

    def gather_wait(b):
        pltpu.make_async_copy(s_hbm.at[pl.ds(0, CHUNK)], srows.at[b], sem_s.at[b]).wait()
        pltpu.make_async_copy(t_hbm.at[pl.ds(0, CHUNK)], trows.at[b], sem_t.at[b]).wait()

    def compute(ci, b):
        off = ci * CHUNK

        def group_body(gi, carry2):
            e0 = gi * 16
            vec = jnp.zeros((16,), jnp.float32)
            for j in range(16):
                e = e0 + j
                acc = jnp.zeros((16,), jnp.float32)
                for k in range(WORDS // 16):
                    sw = plsc.bitcast(srows[b, e, pl.ds(k * 16, 16)], jnp.bfloat16)
                    tw = plsc.bitcast(trows[b, e, pl.ds(k * 16, 16)], jnp.bfloat16)
                    plo, phi = plsc.unpack(sw * tw, format=plsc.PackFormat.INTERLEAVED)
                    acc = acc + plo + phi
                vec = jnp.where(lanes == j, jnp.sum(acc), vec)
            outv[pl.ds(off + e0, 16)] = vec
            return carry2

        lax.fori_loop(0, GROUPS, group_body, 0)

    gather_start(0, 0)
    gather_start(1, 1)

    def chunk_loop(ci, carry):
        b = ci % 2
        gather_wait(b)
        compute(ci, b)

        @pl.when(ci + 2 < NUM_CHUNKS)
        def _():
            gather_start(ci + 2, b)
        return carry

    lax.fori_loop(0, NUM_CHUNKS, chunk_loop, 0)
    pltpu.sync_copy(outv, out_hbm.at[pl.ds(base, EDGES_PER_WORKER)])


@functools.partial(jax.jit)
def kernel(s, t, edge_index):
    ei = edge_index.astype(jnp.int32)
    mesh = plsc.VectorSubcoreMesh(core_axis_name="c", subcore_axis_name="s")
    params = pltpu.CompilerParams(needs_layout_passes=False,
                                  use_tc_tiling_on_sc=False)
    pack = pl.kernel(
        _pack_body,
        out_type=(jax.ShapeDtypeStruct((N_NODES, WORDS), jnp.int32),
                  jax.ShapeDtypeStruct((N_NODES, WORDS), jnp.int32)),
        mesh=mesh,
        compiler_params=params,
        scratch_types=[
            pltpu.VMEM((ROWS_PER_PACKER, D_FEAT), jnp.float32),
            pltpu.VMEM((ROWS_PER_PACKER, WORDS), jnp.int32),
            pltpu.SemaphoreType.DMA,
        ],
    )
    sp, tp = pack(s, t)
    run = pl.kernel(
        _decoder_body,
        out_type=jax.ShapeDtypeStruct((N_EDGES,), jnp.float32),
        mesh=mesh,
        compiler_params=params,
        scratch_types=[
            pltpu.VMEM((EDGES_PER_WORKER,), jnp.int32),
            pltpu.VMEM((EDGES_PER_WORKER,), jnp.int32),
            pltpu.VMEM((2, CHUNK, WORDS), jnp.int32),
            pltpu.VMEM((2, CHUNK, WORDS), jnp.int32),
            pltpu.VMEM((EDGES_PER_WORKER,), jnp.float32),
            pltpu.SemaphoreType.DMA((2,)),
            pltpu.SemaphoreType.DMA((2,)),
        ],
    )
    return run(sp, tp, ei)
